# baseline (speedup 1.0000x reference)
import functools

import jax
import jax.numpy as jnp
from jax.experimental import pallas as pl
from jax.experimental.pallas import tpu as pltpu
from jax.experimental.pallas import tpu_sc as plsc

N = 10000
NPAD = 10240
E = 320000
EPAD = 327680
GROUPS = EPAD // 128
NC = 2
NS = 16
GPW = GROUPS // (NC * NS)
CHUNK_G = 8
NCHUNK = GPW // CHUNK_G
D = 16

_MESH = plsc.VectorSubcoreMesh(core_axis_name="c", subcore_axis_name="s")


def _make_sc_agg(compute_norm):
    out_type = [jax.ShapeDtypeStruct((NC, NPAD, D), jnp.float32)]
    if compute_norm:
        out_type.append(jax.ShapeDtypeStruct((GROUPS, 128), jnp.float32))
        out_type.append(jax.ShapeDtypeStruct((NPAD,), jnp.float32))
    scratch = [
        pltpu.VMEM_SHARED((NPAD, D), jnp.float32),
        pltpu.VMEM((CHUNK_G, 128), jnp.int32),
        pltpu.VMEM((CHUNK_G, 128), jnp.int32),
        pltpu.VMEM((CHUNK_G, 128), jnp.float32),
        pltpu.VMEM((CHUNK_G * 128, D), jnp.float32),
        pltpu.SemaphoreType.DMA,
        pltpu.SemaphoreType.DMA,
        pltpu.SemaphoreType.DMA,
        pltpu.SemaphoreType.DMA,
        pltpu.VMEM_SHARED((NPAD, D), jnp.float32),
    ]
    if compute_norm:
        scratch.append(pltpu.VMEM((NPAD,), jnp.float32))
        scratch.append(pltpu.VMEM_SHARED((NPAD,), jnp.float32))
        scratch.append(pltpu.VMEM((640,), jnp.float32))
        scratch.append(pltpu.VMEM((GPW * 2, 128), jnp.int32))
        scratch.append(pltpu.VMEM((GPW * 2, 128), jnp.float32))

    def body(*refs):
        if compute_norm:
            (row_hbm, col_hbm, ew_hbm, h_hbm,
             out_hbm, norm_hbm, dis_hbm, acc, row_v, col_v, nv, rows,
             sem_i, sem_g, sem_s, sem_w, h_sh, dis_v, deg_sh, dslice_v,
             colw, eww) = refs
        else:
            (row_hbm, col_hbm, norm_in_hbm, h_hbm,
             out_hbm, acc, row_v, col_v, nv, rows,
             sem_i, sem_g, sem_s, sem_w, h_sh) = refs
        cid = jax.lax.axis_index("c")
        sid = jax.lax.axis_index("s")
        wid = sid * NC + cid

        def _zero(i, carry):
            rows[i, :] = jnp.zeros((16,), jnp.float32)
            return carry

        jax.lax.fori_loop(0, 640, _zero, 0)
        pltpu.sync_copy(rows.at[pl.ds(0, 640)], acc.at[pl.ds(sid * 640, 640)])
        pltpu.sync_copy(h_hbm.at[pl.ds(sid * 640, 640)],
                        rows.at[pl.ds(0, 640)])
        pltpu.sync_copy(rows.at[pl.ds(0, 640)], h_sh.at[pl.ds(sid * 640, 640)])
        if compute_norm:
            nsl = pl.ds(sid * 640, 640)
            dsl = pl.ds(sid * 2 * GPW, 2 * GPW)
            ld_c = pltpu.async_copy(col_hbm.at[dsl], colw, sem_i)
            ld_e = pltpu.async_copy(ew_hbm.at[dsl], eww, sem_i)

            def _zero1(i, carry):
                dslice_v[pl.ds(i * 16, 16)] = jnp.zeros((16,), jnp.float32)
                return carry

            jax.lax.fori_loop(0, 40, _zero1, 0)
            pltpu.sync_copy(dslice_v, deg_sh.at[nsl])
            ld_c.wait()
            ld_e.wait()
            plsc.subcore_barrier()

            def _degscat(g, carry):
                pltpu.async_copy(eww.at[g], deg_sh.at[colw.at[g]],
                                 sem_s, add=True)
                return carry

            jax.lax.fori_loop(0, 2 * GPW, _degscat, 0)

            def _degdrain(g, carry):
                pltpu.make_async_copy(eww.at[0], deg_sh.at[colw.at[0]],
                                      sem_s).wait()
                return carry

            jax.lax.fori_loop(0, 2 * GPW, _degdrain, 0)
            plsc.subcore_barrier()

            pltpu.sync_copy(deg_sh.at[nsl], dslice_v)

            def _dis(i, carry):
                sl = pl.ds(i * 16, 16)
                x = 1.0 + dslice_v[sl]
                bits = plsc.bitcast(x, jnp.int32)
                y = plsc.bitcast(
                    jnp.int32(0x5F3759DF) - (bits >> 1), jnp.float32)
                for _ in range(3):
                    y = y * (1.5 - 0.5 * x * y * y)
                dis_v[sl] = y
                return carry

            jax.lax.fori_loop(0, 40, _dis, 0)
            pltpu.sync_copy(dis_v.at[pl.ds(0, 640)], deg_sh.at[nsl])

            @pl.when(cid == 0)
            def _():
                pltpu.sync_copy(dis_v.at[pl.ds(0, 640)], dis_hbm.at[nsl])

            plsc.subcore_barrier()
            pltpu.sync_copy(deg_sh, dis_v)
        plsc.subcore_barrier()

        def _chunk(c, carry):
            goff = wid * GPW + c * CHUNK_G
            sl_g = pl.ds(goff, CHUNK_G)
            loads = [pltpu.async_copy(row_hbm.at[sl_g], row_v, sem_i),
                     pltpu.async_copy(col_hbm.at[sl_g], col_v, sem_i)]
            if compute_norm:
                loads.append(pltpu.async_copy(ew_hbm.at[sl_g], nv, sem_i))
            else:
                loads.append(
                    pltpu.async_copy(norm_in_hbm.at[sl_g], nv, sem_i))
            for cp in loads:
                cp.wait()

            gathers = [
                pltpu.async_copy(h_sh.at[row_v.at[g]],
                                 rows.at[pl.ds(g * 128, 128)], sem_g)
                for g in range(CHUNK_G)
            ]

            norm_wb = None
            if compute_norm:
                def _norm16(j, carry2):
                    gi = j // 8
                    sl = pl.ds((j % 8) * 16, 16)
                    r16 = row_v[gi, sl]
                    c16 = col_v[gi, sl]
                    n16 = (plsc.load_gather(dis_v, [r16]) * nv[gi, sl]
                           * plsc.load_gather(dis_v, [c16]))
                    nv[gi, sl] = n16
                    return carry2

                jax.lax.fori_loop(0, CHUNK_G * 8, _norm16, 0)
                norm_wb = pltpu.async_copy(nv, norm_hbm.at[sl_g], sem_w)

            scatters = []
            for g in range(CHUNK_G):
                gathers[g].wait()

                def _scale16(j, carry2, g=g):
                    n16 = nv[g, pl.ds(j * 16, 16)]
                    base = g * 128 + j * 16
                    for e in range(16):
                        rows[base + e, :] = rows[base + e, :] * n16[e]
                    return carry2

                jax.lax.fori_loop(0, 8, _scale16, 0)
                scatters.append(
                    pltpu.async_copy(rows.at[pl.ds(g * 128, 128)],
                                     acc.at[col_v.at[g]], sem_s, add=True))
            for cp in scatters:
                cp.wait()
            if norm_wb is not None:
                norm_wb.wait()
            return carry

        jax.lax.fori_loop(0, NCHUNK, _chunk, 0)
        plsc.subcore_barrier()

        pltpu.sync_copy(acc.at[pl.ds(sid * 640, 640)], rows.at[pl.ds(0, 640)])
        pltpu.sync_copy(rows.at[pl.ds(0, 640)],
                        out_hbm.at[cid, pl.ds(sid * 640, 640)])

    return pl.kernel(body, out_type=out_type, mesh=_MESH,
                     scratch_types=scratch,
                     compiler_params=pltpu.CompilerParams(
                         needs_layout_passes=False,
                         use_tc_tiling_on_sc=False))


_sc_agg_l1 = _make_sc_agg(True)
_sc_agg_l2 = _make_sc_agg(False)


def _tc_mm1_body(x_ref, w_ref, h_ref):
    h_ref[...] = jnp.dot(x_ref[...], w_ref[...],
                         preferred_element_type=jnp.float32,
                         precision=jax.lax.Precision.HIGHEST)


def _tc_mm1(x, W1):
    return pl.pallas_call(
        _tc_mm1_body,
        grid=(10,),
        in_specs=[
            pl.BlockSpec((1024, 128), lambda i: (i, 0)),
            pl.BlockSpec((128, D), lambda i: (0, 0)),
        ],
        out_specs=pl.BlockSpec((1024, D), lambda i: (i, 0)),
        out_shape=jax.ShapeDtypeStruct((NPAD, D), jnp.float32),
    )(x, W1)


def _tc_mid_body(aggp_ref, h1_ref, dis_ref, b1_ref, w2_ref, h2_ref):
    sn = dis_ref[...] * dis_ref[...]
    a = aggp_ref[0] + aggp_ref[1] + h1_ref[...] * sn + b1_ref[...]
    z = jnp.maximum(a, 0.0)
    h2_ref[...] = jnp.dot(z, w2_ref[...],
                          preferred_element_type=jnp.float32,
                          precision=jax.lax.Precision.HIGHEST)


def _tc_mid(aggp, h1, dis, b1, W2):
    return pl.pallas_call(
        _tc_mid_body,
        grid=(10,),
        in_specs=[
            pl.BlockSpec((NC, 1024, D), lambda i: (0, i, 0)),
            pl.BlockSpec((1024, D), lambda i: (i, 0)),
            pl.BlockSpec((1024, 1), lambda i: (i, 0)),
            pl.BlockSpec((1, D), lambda i: (0, 0)),
            pl.BlockSpec((D, D), lambda i: (0, 0)),
        ],
        out_specs=pl.BlockSpec((1024, D), lambda i: (i, 0)),
        out_shape=jax.ShapeDtypeStruct((NPAD, D), jnp.float32),
    )(aggp, h1, dis, b1, W2)


def _tc_final_body(aggp_ref, h2_ref, dis_ref, b2_ref, final_ref, lsm_ref):
    sn = dis_ref[...] * dis_ref[...]
    f = aggp_ref[0] + aggp_ref[1] + h2_ref[...] * sn + b2_ref[...]
    final_ref[...] = f
    m = jnp.max(f, axis=1, keepdims=True)
    lse = jnp.log(jnp.sum(jnp.exp(f - m), axis=1, keepdims=True))
    lsm_ref[...] = f - m - lse


def _tc_final(aggp, h2, dis, b2):
    return pl.pallas_call(
        _tc_final_body,
        grid=(10,),
        in_specs=[
            pl.BlockSpec((NC, 1000, D), lambda i: (0, i, 0)),
            pl.BlockSpec((1000, D), lambda i: (i, 0)),
            pl.BlockSpec((1000, 1), lambda i: (i, 0)),
            pl.BlockSpec((1, D), lambda i: (0, 0)),
        ],
        out_specs=[
            pl.BlockSpec((1000, D), lambda i: (i, 0)),
            pl.BlockSpec((1000, D), lambda i: (i, 0)),
        ],
        out_shape=[
            jax.ShapeDtypeStruct((N, D), jnp.float32),
            jax.ShapeDtypeStruct((N, D), jnp.float32),
        ],
    )(aggp, h2, dis, b2)


@jax.jit
def kernel(x, edge_index, edge_weight, W1, b1, W2, b2):
    pad = EPAD - E
    rowp = jnp.concatenate(
        [edge_index[0], jnp.zeros((pad,), jnp.int32)]).reshape(GROUPS, 128)
    colp = jnp.concatenate(
        [edge_index[1], jnp.zeros((pad,), jnp.int32)]).reshape(GROUPS, 128)
    ewp = jnp.concatenate(
        [edge_weight, jnp.zeros((pad,), jnp.float32)]).reshape(GROUPS, 128)

    xp = jnp.concatenate(
        [x, jnp.zeros((NPAD - N, x.shape[1]), jnp.float32)])
    h1 = _tc_mm1(xp, W1)
    aggp1, normp, dis = _sc_agg_l1(rowp, colp, ewp, h1)
    dis = dis.reshape(NPAD, 1)
    h2 = _tc_mid(aggp1, h1, dis, b1.reshape(1, D), W2)
    (aggp2,) = _sc_agg_l2(rowp, colp, normp, h2)
    final, lsm = _tc_final(aggp2, h2, dis, b2.reshape(1, D))
    return (final, lsm)


# speedup vs baseline: 1.0745x; 1.0745x over previous
import functools

import jax
import jax.numpy as jnp
from jax.experimental import pallas as pl
from jax.experimental.pallas import tpu as pltpu
from jax.experimental.pallas import tpu_sc as plsc

N = 10000
NPAD = 10240
E = 320000
EPAD = 327680
GROUPS = EPAD // 128
NC = 2
NS = 16
GPW = GROUPS // (NC * NS)
CHUNK_G = 16
NCHUNK = GPW // CHUNK_G
D = 16

_MESH = plsc.VectorSubcoreMesh(core_axis_name="c", subcore_axis_name="s")


@functools.partial(
    pl.kernel,
    out_type=jax.ShapeDtypeStruct((NC, NPAD), jnp.float32),
    mesh=_MESH,
    scratch_types=[
        pltpu.VMEM_SHARED((NPAD,), jnp.float32),
        pltpu.VMEM((GPW, 128), jnp.int32),
        pltpu.VMEM((GPW, 128), jnp.float32),
        pltpu.VMEM((640,), jnp.float32),
        pltpu.SemaphoreType.DMA,
        pltpu.SemaphoreType.DMA,
    ],
)
def _sc_degree(col_hbm, ew_hbm, out_hbm, acc, col_v, ew_v, zbuf, sem_i,
               sem_s):
    cid = jax.lax.axis_index("c")
    sid = jax.lax.axis_index("s")
    wid = sid * NC + cid

    sl_w = pl.ds(wid * GPW, GPW)
    ld_c = pltpu.async_copy(col_hbm.at[sl_w], col_v, sem_i)
    ld_e = pltpu.async_copy(ew_hbm.at[sl_w], ew_v, sem_i)

    def _zero(i, carry):
        zbuf[pl.ds(i * 16, 16)] = jnp.zeros((16,), jnp.float32)
        return carry

    jax.lax.fori_loop(0, 40, _zero, 0)
    pltpu.sync_copy(zbuf, acc.at[pl.ds(sid * 640, 640)])
    ld_c.wait()
    ld_e.wait()
    plsc.subcore_barrier()

    scatters = [
        pltpu.async_copy(ew_v.at[g], acc.at[col_v.at[g]], sem_s, add=True)
        for g in range(GPW)
    ]
    for cp in scatters:
        cp.wait()
    plsc.subcore_barrier()

    pltpu.sync_copy(acc.at[pl.ds(sid * 640, 640)], zbuf)
    pltpu.sync_copy(zbuf, out_hbm.at[cid, pl.ds(sid * 640, 640)])


def _make_sc_agg(compute_norm):
    out_type = [jax.ShapeDtypeStruct((NC, NPAD, D), jnp.float32)]
    if compute_norm:
        out_type.append(jax.ShapeDtypeStruct((GROUPS, 128), jnp.float32))
        out_type.append(jax.ShapeDtypeStruct((NPAD,), jnp.float32))
    scratch = [
        pltpu.VMEM_SHARED((NPAD, D), jnp.float32),
        pltpu.VMEM((CHUNK_G, 128), jnp.int32),
        pltpu.VMEM((CHUNK_G, 128), jnp.int32),
        pltpu.VMEM((CHUNK_G, 128), jnp.float32),
        pltpu.VMEM((CHUNK_G * 128, D), jnp.float32),
        pltpu.SemaphoreType.DMA,
        pltpu.SemaphoreType.DMA,
        pltpu.SemaphoreType.DMA,
        pltpu.SemaphoreType.DMA,
        pltpu.VMEM_SHARED((NPAD, D), jnp.float32),
    ]
    if compute_norm:
        scratch.append(pltpu.VMEM((NPAD,), jnp.float32))
        scratch.append(pltpu.VMEM_SHARED((NPAD,), jnp.float32))
        scratch.append(pltpu.VMEM((2, 640), jnp.float32))

    def body(*refs):
        if compute_norm:
            (row_hbm, col_hbm, ew_hbm, degp_hbm, h_hbm,
             out_hbm, norm_hbm, dis_hbm, acc, row_v, col_v, nv, rows,
             sem_i, sem_g, sem_s, sem_w, h_sh, dis_v, dis_sh, degp_v) = refs
        else:
            (row_hbm, col_hbm, norm_in_hbm, h_hbm,
             out_hbm, acc, row_v, col_v, nv, rows,
             sem_i, sem_g, sem_s, sem_w, h_sh) = refs
        cid = jax.lax.axis_index("c")
        sid = jax.lax.axis_index("s")
        wid = sid * NC + cid

        def _zero(i, carry):
            rows[i, :] = jnp.zeros((16,), jnp.float32)
            return carry

        jax.lax.fori_loop(0, 640, _zero, 0)
        pltpu.sync_copy(rows.at[pl.ds(0, 640)], acc.at[pl.ds(sid * 640, 640)])
        pltpu.sync_copy(h_hbm.at[pl.ds(sid * 640, 640)],
                        rows.at[pl.ds(0, 640)])
        pltpu.sync_copy(rows.at[pl.ds(0, 640)], h_sh.at[pl.ds(sid * 640, 640)])
        if compute_norm:
            nsl = pl.ds(sid * 640, 640)
            pltpu.sync_copy(degp_hbm.at[0, nsl], degp_v.at[0])
            pltpu.sync_copy(degp_hbm.at[1, nsl], degp_v.at[1])

            def _dis(i, carry):
                sl = pl.ds(i * 16, 16)
                x = 1.0 + degp_v[0, sl] + degp_v[1, sl]
                bits = plsc.bitcast(x, jnp.int32)
                y = plsc.bitcast(
                    jnp.int32(0x5F3759DF) - (bits >> 1), jnp.float32)
                for _ in range(3):
                    y = y * (1.5 - 0.5 * x * y * y)
                dis_v[sl] = y
                return carry

            jax.lax.fori_loop(0, 40, _dis, 0)
            pltpu.sync_copy(dis_v.at[pl.ds(0, 640)], dis_sh.at[nsl])

            @pl.when(cid == 0)
            def _():
                pltpu.sync_copy(dis_v.at[pl.ds(0, 640)], dis_hbm.at[nsl])

            plsc.subcore_barrier()
            pltpu.sync_copy(dis_sh, dis_v)
        plsc.subcore_barrier()

        def _chunk(c, carry):
            goff = wid * GPW + c * CHUNK_G
            sl_g = pl.ds(goff, CHUNK_G)
            loads = [pltpu.async_copy(row_hbm.at[sl_g], row_v, sem_i),
                     pltpu.async_copy(col_hbm.at[sl_g], col_v, sem_i)]
            if compute_norm:
                loads.append(pltpu.async_copy(ew_hbm.at[sl_g], nv, sem_i))
            else:
                loads.append(
                    pltpu.async_copy(norm_in_hbm.at[sl_g], nv, sem_i))
            for cp in loads:
                cp.wait()

            gathers = [
                pltpu.async_copy(h_sh.at[row_v.at[g]],
                                 rows.at[pl.ds(g * 128, 128)], sem_g)
                for g in range(CHUNK_G)
            ]

            norm_wb = None
            if compute_norm:
                def _norm16(j, carry2):
                    gi = j // 8
                    sl = pl.ds((j % 8) * 16, 16)
                    r16 = row_v[gi, sl]
                    c16 = col_v[gi, sl]
                    n16 = (plsc.load_gather(dis_v, [r16]) * nv[gi, sl]
                           * plsc.load_gather(dis_v, [c16]))
                    nv[gi, sl] = n16
                    return carry2

                jax.lax.fori_loop(0, CHUNK_G * 8, _norm16, 0)
                norm_wb = pltpu.async_copy(nv, norm_hbm.at[sl_g], sem_w)

            scatters = []
            for g in range(CHUNK_G):
                gathers[g].wait()

                def _scale16(j, carry2, g=g):
                    n16 = nv[g, pl.ds(j * 16, 16)]
                    base = g * 128 + j * 16
                    for e in range(16):
                        rows[base + e, :] = rows[base + e, :] * n16[e]
                    return carry2

                jax.lax.fori_loop(0, 8, _scale16, 0)
                scatters.append(
                    pltpu.async_copy(rows.at[pl.ds(g * 128, 128)],
                                     acc.at[col_v.at[g]], sem_s, add=True))
            for cp in scatters:
                cp.wait()
            if norm_wb is not None:
                norm_wb.wait()
            return carry

        jax.lax.fori_loop(0, NCHUNK, _chunk, 0)
        plsc.subcore_barrier()

        pltpu.sync_copy(acc.at[pl.ds(sid * 640, 640)], rows.at[pl.ds(0, 640)])
        pltpu.sync_copy(rows.at[pl.ds(0, 640)],
                        out_hbm.at[cid, pl.ds(sid * 640, 640)])

    return pl.kernel(body, out_type=out_type, mesh=_MESH,
                     scratch_types=scratch,
                     compiler_params=pltpu.CompilerParams(
                         needs_layout_passes=False,
                         use_tc_tiling_on_sc=False))


_sc_agg_l1 = _make_sc_agg(True)
_sc_agg_l2 = _make_sc_agg(False)


def _tc_mm1_body(x_ref, w_ref, h_ref):
    h_ref[...] = jnp.dot(x_ref[...], w_ref[...],
                         preferred_element_type=jnp.float32,
                         precision=jax.lax.Precision.HIGHEST)


def _tc_mm1(x, W1):
    return pl.pallas_call(
        _tc_mm1_body,
        grid=(10,),
        in_specs=[
            pl.BlockSpec((1024, 128), lambda i: (i, 0)),
            pl.BlockSpec((128, D), lambda i: (0, 0)),
        ],
        out_specs=pl.BlockSpec((1024, D), lambda i: (i, 0)),
        out_shape=jax.ShapeDtypeStruct((NPAD, D), jnp.float32),
    )(x, W1)


def _tc_mid_body(aggp_ref, h1_ref, dis_ref, b1_ref, w2_ref, h2_ref):
    sn = dis_ref[...] * dis_ref[...]
    a = aggp_ref[0] + aggp_ref[1] + h1_ref[...] * sn + b1_ref[...]
    z = jnp.maximum(a, 0.0)
    h2_ref[...] = jnp.dot(z, w2_ref[...],
                          preferred_element_type=jnp.float32,
                          precision=jax.lax.Precision.HIGHEST)


def _tc_mid(aggp, h1, dis, b1, W2):
    return pl.pallas_call(
        _tc_mid_body,
        grid=(10,),
        in_specs=[
            pl.BlockSpec((NC, 1024, D), lambda i: (0, i, 0)),
            pl.BlockSpec((1024, D), lambda i: (i, 0)),
            pl.BlockSpec((1024, 1), lambda i: (i, 0)),
            pl.BlockSpec((1, D), lambda i: (0, 0)),
            pl.BlockSpec((D, D), lambda i: (0, 0)),
        ],
        out_specs=pl.BlockSpec((1024, D), lambda i: (i, 0)),
        out_shape=jax.ShapeDtypeStruct((NPAD, D), jnp.float32),
    )(aggp, h1, dis, b1, W2)


def _tc_final_body(aggp_ref, h2_ref, dis_ref, b2_ref, final_ref, lsm_ref):
    sn = dis_ref[...] * dis_ref[...]
    f = aggp_ref[0] + aggp_ref[1] + h2_ref[...] * sn + b2_ref[...]
    final_ref[...] = f
    m = jnp.max(f, axis=1, keepdims=True)
    lse = jnp.log(jnp.sum(jnp.exp(f - m), axis=1, keepdims=True))
    lsm_ref[...] = f - m - lse


def _tc_final(aggp, h2, dis, b2):
    return pl.pallas_call(
        _tc_final_body,
        grid=(10,),
        in_specs=[
            pl.BlockSpec((NC, 1000, D), lambda i: (0, i, 0)),
            pl.BlockSpec((1000, D), lambda i: (i, 0)),
            pl.BlockSpec((1000, 1), lambda i: (i, 0)),
            pl.BlockSpec((1, D), lambda i: (0, 0)),
        ],
        out_specs=[
            pl.BlockSpec((1000, D), lambda i: (i, 0)),
            pl.BlockSpec((1000, D), lambda i: (i, 0)),
        ],
        out_shape=[
            jax.ShapeDtypeStruct((N, D), jnp.float32),
            jax.ShapeDtypeStruct((N, D), jnp.float32),
        ],
    )(aggp, h2, dis, b2)


@jax.jit
def kernel(x, edge_index, edge_weight, W1, b1, W2, b2):
    pad = EPAD - E
    rowp = jnp.concatenate(
        [edge_index[0], jnp.zeros((pad,), jnp.int32)]).reshape(GROUPS, 128)
    colp = jnp.concatenate(
        [edge_index[1], jnp.zeros((pad,), jnp.int32)]).reshape(GROUPS, 128)
    ewp = jnp.concatenate(
        [edge_weight, jnp.zeros((pad,), jnp.float32)]).reshape(GROUPS, 128)

    xp = jnp.concatenate(
        [x, jnp.zeros((NPAD - N, x.shape[1]), jnp.float32)])
    degp = _sc_degree(colp, ewp)
    h1 = _tc_mm1(xp, W1)
    aggp1, normp, dis = _sc_agg_l1(rowp, colp, ewp, degp, h1)
    dis = dis.reshape(NPAD, 1)
    h2 = _tc_mid(aggp1, h1, dis, b1.reshape(1, D), W2)
    (aggp2,) = _sc_agg_l2(rowp, colp, normp, h2)
    final, lsm = _tc_final(aggp2, h2, dis, b2.reshape(1, D))
    return (final, lsm)


# speedup vs baseline: 1.1246x; 1.0466x over previous
import functools

import jax
import jax.numpy as jnp
from jax.experimental import pallas as pl
from jax.experimental.pallas import tpu as pltpu
from jax.experimental.pallas import tpu_sc as plsc

N = 10000
NPAD = 10240
E = 320000
EPAD = 327680
GROUPS = EPAD // 128
NC = 2
NS = 16
GPW = GROUPS // (NC * NS)
CHUNK_G = 16
NCHUNK = GPW // CHUNK_G
D = 16

_MESH = plsc.VectorSubcoreMesh(core_axis_name="c", subcore_axis_name="s")


@functools.partial(
    pl.kernel,
    out_type=jax.ShapeDtypeStruct((NC, NPAD), jnp.float32),
    mesh=_MESH,
    scratch_types=[
        pltpu.VMEM_SHARED((NPAD,), jnp.float32),
        pltpu.VMEM((GPW, 128), jnp.int32),
        pltpu.VMEM((GPW, 128), jnp.float32),
        pltpu.VMEM((640,), jnp.float32),
        pltpu.SemaphoreType.DMA,
        pltpu.SemaphoreType.DMA,
    ],
)
def _sc_degree(col_hbm, ew_hbm, out_hbm, acc, col_v, ew_v, zbuf, sem_i,
               sem_s):
    cid = jax.lax.axis_index("c")
    sid = jax.lax.axis_index("s")
    wid = sid * NC + cid

    sl_w = pl.ds(wid * GPW, GPW)
    ld_c = pltpu.async_copy(col_hbm.at[sl_w], col_v, sem_i)
    ld_e = pltpu.async_copy(ew_hbm.at[sl_w], ew_v, sem_i)

    def _zero(i, carry):
        zbuf[pl.ds(i * 16, 16)] = jnp.zeros((16,), jnp.float32)
        return carry

    jax.lax.fori_loop(0, 40, _zero, 0)
    pltpu.sync_copy(zbuf, acc.at[pl.ds(sid * 640, 640)])
    ld_c.wait()
    ld_e.wait()
    plsc.subcore_barrier()

    scatters = [
        pltpu.async_copy(ew_v.at[g], acc.at[col_v.at[g]], sem_s, add=True)
        for g in range(GPW)
    ]
    for cp in scatters:
        cp.wait()
    plsc.subcore_barrier()

    pltpu.sync_copy(acc.at[pl.ds(sid * 640, 640)], zbuf)
    pltpu.sync_copy(zbuf, out_hbm.at[cid, pl.ds(sid * 640, 640)])


def _make_sc_agg(compute_norm):
    out_type = [jax.ShapeDtypeStruct((NC, NPAD, D), jnp.float32)]
    if compute_norm:
        out_type.append(jax.ShapeDtypeStruct((GROUPS, 128), jnp.float32))
        out_type.append(jax.ShapeDtypeStruct((NPAD,), jnp.float32))
    scratch = [
        pltpu.VMEM_SHARED((NPAD, D), jnp.float32),
        pltpu.VMEM((CHUNK_G, 128), jnp.int32),
        pltpu.VMEM((CHUNK_G, 128), jnp.int32),
        pltpu.VMEM((CHUNK_G, 128), jnp.float32),
        pltpu.VMEM((CHUNK_G * 128, D), jnp.float32),
        pltpu.SemaphoreType.DMA,
        pltpu.SemaphoreType.DMA,
        pltpu.SemaphoreType.DMA,
        pltpu.SemaphoreType.DMA,
        pltpu.VMEM_SHARED((NPAD, D), jnp.float32),
    ]
    if compute_norm:
        scratch.append(pltpu.VMEM((NPAD,), jnp.float32))
        scratch.append(pltpu.VMEM_SHARED((NPAD,), jnp.float32))
        scratch.append(pltpu.VMEM((2, 640), jnp.float32))

    def body(*refs):
        if compute_norm:
            (row_hbm, col_hbm, ew_hbm, degp_hbm, h_hbm,
             out_hbm, norm_hbm, dis_hbm, acc, row_v, col_v, nv, rows,
             sem_i, sem_g, sem_s, sem_w, h_sh, dis_v, dis_sh, degp_v) = refs
        else:
            (row_hbm, col_hbm, norm_in_hbm, h_hbm,
             out_hbm, acc, row_v, col_v, nv, rows,
             sem_i, sem_g, sem_s, sem_w, h_sh) = refs
        cid = jax.lax.axis_index("c")
        sid = jax.lax.axis_index("s")
        wid = sid * NC + cid

        def _zero(i, carry):
            rows[i, :] = jnp.zeros((16,), jnp.float32)
            return carry

        jax.lax.fori_loop(0, 640, _zero, 0)
        pltpu.sync_copy(rows.at[pl.ds(0, 640)], acc.at[pl.ds(sid * 640, 640)])
        pltpu.sync_copy(h_hbm.at[pl.ds(sid * 640, 640)],
                        rows.at[pl.ds(0, 640)])
        pltpu.sync_copy(rows.at[pl.ds(0, 640)], h_sh.at[pl.ds(sid * 640, 640)])
        if compute_norm:
            nsl = pl.ds(sid * 640, 640)
            pltpu.sync_copy(degp_hbm.at[0, nsl], degp_v.at[0])
            pltpu.sync_copy(degp_hbm.at[1, nsl], degp_v.at[1])

            def _dis(i, carry):
                sl = pl.ds(i * 16, 16)
                x = 1.0 + degp_v[0, sl] + degp_v[1, sl]
                bits = plsc.bitcast(x, jnp.int32)
                y = plsc.bitcast(
                    jnp.int32(0x5F3759DF) - (bits >> 1), jnp.float32)
                for _ in range(3):
                    y = y * (1.5 - 0.5 * x * y * y)
                dis_v[sl] = y
                return carry

            jax.lax.fori_loop(0, 40, _dis, 0)
            pltpu.sync_copy(dis_v.at[pl.ds(0, 640)], dis_sh.at[nsl])

            @pl.when(cid == 0)
            def _():
                pltpu.sync_copy(dis_v.at[pl.ds(0, 640)], dis_hbm.at[nsl])

            plsc.subcore_barrier()
            pltpu.sync_copy(dis_sh, dis_v)
        plsc.subcore_barrier()

        def _chunk(c, carry):
            goff = wid * GPW + c * CHUNK_G
            sl_g = pl.ds(goff, CHUNK_G)
            loads = [pltpu.async_copy(row_hbm.at[sl_g], row_v, sem_i),
                     pltpu.async_copy(col_hbm.at[sl_g], col_v, sem_i)]
            if compute_norm:
                loads.append(pltpu.async_copy(ew_hbm.at[sl_g], nv, sem_i))
            else:
                loads.append(
                    pltpu.async_copy(norm_in_hbm.at[sl_g], nv, sem_i))
            for cp in loads:
                cp.wait()

            gathers = [
                pltpu.async_copy(h_sh.at[row_v.at[g]],
                                 rows.at[pl.ds(g * 128, 128)], sem_g)
                for g in range(CHUNK_G)
            ]

            norm_wb = None
            if compute_norm:
                @functools.partial(plsc.parallel_loop, 0, CHUNK_G * 8,
                                   unroll=4)
                def _norm16(j):
                    gi = j // 8
                    sl = pl.ds((j % 8) * 16, 16)
                    r16 = row_v[gi, sl]
                    c16 = col_v[gi, sl]
                    n16 = (plsc.load_gather(dis_v, [r16]) * nv[gi, sl]
                           * plsc.load_gather(dis_v, [c16]))
                    nv[gi, sl] = n16
                norm_wb = pltpu.async_copy(nv, norm_hbm.at[sl_g], sem_w)

            scatters = []
            for g in range(CHUNK_G):
                gathers[g].wait()

                @functools.partial(plsc.parallel_loop, 0, 8, unroll=2)
                def _scale16(j, g=g):
                    n16 = nv[g, pl.ds(j * 16, 16)]
                    base = g * 128 + j * 16
                    for e in range(16):
                        rows[base + e, :] = rows[base + e, :] * n16[e]
                scatters.append(
                    pltpu.async_copy(rows.at[pl.ds(g * 128, 128)],
                                     acc.at[col_v.at[g]], sem_s, add=True))
            for cp in scatters:
                cp.wait()
            if norm_wb is not None:
                norm_wb.wait()
            return carry

        jax.lax.fori_loop(0, NCHUNK, _chunk, 0)
        plsc.subcore_barrier()

        pltpu.sync_copy(acc.at[pl.ds(sid * 640, 640)], rows.at[pl.ds(0, 640)])
        pltpu.sync_copy(rows.at[pl.ds(0, 640)],
                        out_hbm.at[cid, pl.ds(sid * 640, 640)])

    return pl.kernel(body, out_type=out_type, mesh=_MESH,
                     scratch_types=scratch,
                     compiler_params=pltpu.CompilerParams(
                         needs_layout_passes=False,
                         use_tc_tiling_on_sc=False))


_sc_agg_l1 = _make_sc_agg(True)
_sc_agg_l2 = _make_sc_agg(False)


def _tc_mm1_body(x_ref, w_ref, h_ref):
    h_ref[...] = jnp.dot(x_ref[...], w_ref[...],
                         preferred_element_type=jnp.float32,
                         precision=jax.lax.Precision.HIGHEST)


def _tc_mm1(x, W1):
    return pl.pallas_call(
        _tc_mm1_body,
        grid=(10,),
        in_specs=[
            pl.BlockSpec((1024, 128), lambda i: (i, 0)),
            pl.BlockSpec((128, D), lambda i: (0, 0)),
        ],
        out_specs=pl.BlockSpec((1024, D), lambda i: (i, 0)),
        out_shape=jax.ShapeDtypeStruct((NPAD, D), jnp.float32),
    )(x, W1)


def _tc_mid_body(aggp_ref, h1_ref, dis_ref, b1_ref, w2_ref, h2_ref):
    sn = dis_ref[...] * dis_ref[...]
    a = aggp_ref[0] + aggp_ref[1] + h1_ref[...] * sn + b1_ref[...]
    z = jnp.maximum(a, 0.0)
    h2_ref[...] = jnp.dot(z, w2_ref[...],
                          preferred_element_type=jnp.float32,
                          precision=jax.lax.Precision.HIGHEST)


def _tc_mid(aggp, h1, dis, b1, W2):
    return pl.pallas_call(
        _tc_mid_body,
        grid=(10,),
        in_specs=[
            pl.BlockSpec((NC, 1024, D), lambda i: (0, i, 0)),
            pl.BlockSpec((1024, D), lambda i: (i, 0)),
            pl.BlockSpec((1024, 1), lambda i: (i, 0)),
            pl.BlockSpec((1, D), lambda i: (0, 0)),
            pl.BlockSpec((D, D), lambda i: (0, 0)),
        ],
        out_specs=pl.BlockSpec((1024, D), lambda i: (i, 0)),
        out_shape=jax.ShapeDtypeStruct((NPAD, D), jnp.float32),
    )(aggp, h1, dis, b1, W2)


def _tc_final_body(aggp_ref, h2_ref, dis_ref, b2_ref, final_ref, lsm_ref):
    sn = dis_ref[...] * dis_ref[...]
    f = aggp_ref[0] + aggp_ref[1] + h2_ref[...] * sn + b2_ref[...]
    final_ref[...] = f
    m = jnp.max(f, axis=1, keepdims=True)
    lse = jnp.log(jnp.sum(jnp.exp(f - m), axis=1, keepdims=True))
    lsm_ref[...] = f - m - lse


def _tc_final(aggp, h2, dis, b2):
    return pl.pallas_call(
        _tc_final_body,
        grid=(10,),
        in_specs=[
            pl.BlockSpec((NC, 1000, D), lambda i: (0, i, 0)),
            pl.BlockSpec((1000, D), lambda i: (i, 0)),
            pl.BlockSpec((1000, 1), lambda i: (i, 0)),
            pl.BlockSpec((1, D), lambda i: (0, 0)),
        ],
        out_specs=[
            pl.BlockSpec((1000, D), lambda i: (i, 0)),
            pl.BlockSpec((1000, D), lambda i: (i, 0)),
        ],
        out_shape=[
            jax.ShapeDtypeStruct((N, D), jnp.float32),
            jax.ShapeDtypeStruct((N, D), jnp.float32),
        ],
    )(aggp, h2, dis, b2)


@jax.jit
def kernel(x, edge_index, edge_weight, W1, b1, W2, b2):
    pad = EPAD - E
    rowp = jnp.concatenate(
        [edge_index[0], jnp.zeros((pad,), jnp.int32)]).reshape(GROUPS, 128)
    colp = jnp.concatenate(
        [edge_index[1], jnp.zeros((pad,), jnp.int32)]).reshape(GROUPS, 128)
    ewp = jnp.concatenate(
        [edge_weight, jnp.zeros((pad,), jnp.float32)]).reshape(GROUPS, 128)

    xp = jnp.concatenate(
        [x, jnp.zeros((NPAD - N, x.shape[1]), jnp.float32)])
    degp = _sc_degree(colp, ewp)
    h1 = _tc_mm1(xp, W1)
    aggp1, normp, dis = _sc_agg_l1(rowp, colp, ewp, degp, h1)
    dis = dis.reshape(NPAD, 1)
    h2 = _tc_mid(aggp1, h1, dis, b1.reshape(1, D), W2)
    (aggp2,) = _sc_agg_l2(rowp, colp, normp, h2)
    final, lsm = _tc_final(aggp2, h2, dis, b2.reshape(1, D))
    return (final, lsm)
